# baseline (speedup 1.0000x reference)
import functools

import jax
import jax.numpy as jnp
from jax import lax
from jax.experimental import pallas as pl
from jax.experimental.pallas import tpu as pltpu
from jax.experimental.pallas import tpu_sc as plsc

N = 1000000
B = 16384
D = 64
L = 16
NC, NS = 2, 16
NW = NC * NS
BPW = B // NW
CH = 128
NCH = BPW // CH

CBLK = 102400
GRID = (N + CBLK - 1) // CBLK


def _scan_body(tab_ref, w_ref, s1_ref, s2_ref):
    blk = tab_ref[...]
    res = jax.lax.dot_general(
        w_ref[...].T, blk, (((1,), (0,)), ((), ())),
        preferred_element_type=jnp.float32)
    s1_ref[...] = res[0]
    s2_ref[...] = res[1]


_tc_scan = pl.pallas_call(
    _scan_body,
    grid=(GRID,),
    in_specs=[
        pl.BlockSpec((D, CBLK), lambda i: (0, i)),
        pl.BlockSpec((D, 2), lambda i: (0, 0)),
    ],
    out_specs=[
        pl.BlockSpec((CBLK,), lambda i: (i,)),
        pl.BlockSpec((CBLK,), lambda i: (i,)),
    ],
    out_shape=[
        jax.ShapeDtypeStruct((N,), jnp.float32),
        jax.ShapeDtypeStruct((N,), jnp.float32),
    ],
)

_mesh = plsc.VectorSubcoreMesh(core_axis_name="c", subcore_axis_name="s")


@functools.partial(
    pl.kernel,
    out_type=jax.ShapeDtypeStruct((B,), jnp.float32),
    mesh=_mesh,
    compiler_params=pltpu.CompilerParams(needs_layout_passes=False),
    scratch_types=[
        pltpu.VMEM((NCH, CH), jnp.int32),
        pltpu.VMEM((NCH, CH), jnp.int32),
        pltpu.VMEM((BPW,), jnp.float32),
        pltpu.VMEM((BPW,), jnp.float32),
        pltpu.VMEM((L,), jnp.float32),
        pltpu.VMEM((BPW,), jnp.float32),
        pltpu.SemaphoreType.DMA,
    ],
)
def _sc_combine(idx1_hbm, idx2_hbm, s1_hbm, s2_hbm, bias_hbm, out_hbm,
                idx1_v, idx2_v, g1_v, g2_v, b_v, out_v, sem):
    wid = lax.axis_index("s") * NC + lax.axis_index("c")
    base = wid * BPW

    pltpu.sync_copy(bias_hbm, b_v)
    for j in range(NCH):
        pltpu.sync_copy(idx1_hbm.at[pl.ds(base + j * CH, CH)], idx1_v.at[j])
        pltpu.sync_copy(idx2_hbm.at[pl.ds(base + j * CH, CH)], idx2_v.at[j])

    copies = []
    for j in range(NCH):
        copies.append(pltpu.async_copy(
            s1_hbm.at[idx1_v.at[j]], g1_v.at[pl.ds(j * CH, CH)], sem))
        copies.append(pltpu.async_copy(
            s2_hbm.at[idx2_v.at[j]], g2_v.at[pl.ds(j * CH, CH)], sem))
    for c in copies:
        c.wait()

    bvec = b_v[pl.ds(0, L)]
    for k in range(BPW // L):
        out_v[pl.ds(k * L, L)] = (
            g1_v[pl.ds(k * L, L)] + g2_v[pl.ds(k * L, L)] + bvec)

    pltpu.sync_copy(out_v, out_hbm.at[pl.ds(base, BPW)])


def kernel(article1_idx, article2_idx, vector_tensor, W, b):
    tab_t = vector_tensor.T
    wcols = W.astype(jnp.float32).reshape(2, D).T
    s1, s2 = _tc_scan(tab_t, wcols)
    bias = jnp.broadcast_to(b.astype(jnp.float32), (L,))
    out = _sc_combine(article1_idx.astype(jnp.int32),
                      article2_idx.astype(jnp.int32),
                      s1, s2, bias)
    return out.reshape(B, 1)


# speedup vs baseline: 1.0158x; 1.0158x over previous
import functools

import jax
import jax.numpy as jnp
from jax import lax
from jax.experimental import pallas as pl
from jax.experimental.pallas import tpu as pltpu
from jax.experimental.pallas import tpu_sc as plsc

N = 1000000
B = 16384
D = 64
L = 16
NC, NS = 2, 16
NW = NC * NS
BPW = B // NW
CH = 128
NCH = BPW // CH

CBLK = 102400
GRID = (N + CBLK - 1) // CBLK


def _scan_body(top_ref, bot_ref, w_ref, s1_ref, s2_ref):
    wt = w_ref[...]
    dn = (((1,), (0,)), ((), ()))
    res = (jax.lax.dot_general(wt[: D // 2].T, top_ref[...], dn,
                               preferred_element_type=jnp.float32)
           + jax.lax.dot_general(wt[D // 2:].T, bot_ref[...], dn,
                                 preferred_element_type=jnp.float32))
    s1_ref[...] = res[0]
    s2_ref[...] = res[1]


_tc_scan = pl.pallas_call(
    _scan_body,
    grid=(GRID,),
    in_specs=[
        pl.BlockSpec((D // 2, CBLK), lambda i: (0, i)),
        pl.BlockSpec((D // 2, CBLK), lambda i: (1, i)),
        pl.BlockSpec((D, 2), lambda i: (0, 0)),
    ],
    out_specs=[
        pl.BlockSpec((CBLK,), lambda i: (i,)),
        pl.BlockSpec((CBLK,), lambda i: (i,)),
    ],
    out_shape=[
        jax.ShapeDtypeStruct((N,), jnp.float32),
        jax.ShapeDtypeStruct((N,), jnp.float32),
    ],
)

_mesh = plsc.VectorSubcoreMesh(core_axis_name="c", subcore_axis_name="s")


@functools.partial(
    pl.kernel,
    out_type=jax.ShapeDtypeStruct((B,), jnp.float32),
    mesh=_mesh,
    compiler_params=pltpu.CompilerParams(needs_layout_passes=False),
    scratch_types=[
        pltpu.VMEM((NCH, CH), jnp.int32),
        pltpu.VMEM((NCH, CH), jnp.int32),
        pltpu.VMEM((BPW,), jnp.float32),
        pltpu.VMEM((BPW,), jnp.float32),
        pltpu.VMEM((L,), jnp.float32),
        pltpu.VMEM((BPW,), jnp.float32),
        pltpu.SemaphoreType.DMA,
    ],
)
def _sc_combine(idx1_hbm, idx2_hbm, s1_hbm, s2_hbm, bias_hbm, out_hbm,
                idx1_v, idx2_v, g1_v, g2_v, b_v, out_v, sem):
    wid = lax.axis_index("s") * NC + lax.axis_index("c")
    base = wid * BPW

    pltpu.sync_copy(bias_hbm, b_v)
    for j in range(NCH):
        pltpu.sync_copy(idx1_hbm.at[pl.ds(base + j * CH, CH)], idx1_v.at[j])
        pltpu.sync_copy(idx2_hbm.at[pl.ds(base + j * CH, CH)], idx2_v.at[j])

    copies = []
    for j in range(NCH):
        copies.append(pltpu.async_copy(
            s1_hbm.at[idx1_v.at[j]], g1_v.at[pl.ds(j * CH, CH)], sem))
        copies.append(pltpu.async_copy(
            s2_hbm.at[idx2_v.at[j]], g2_v.at[pl.ds(j * CH, CH)], sem))
    for c in copies:
        c.wait()

    bvec = b_v[pl.ds(0, L)]
    for k in range(BPW // L):
        out_v[pl.ds(k * L, L)] = (
            g1_v[pl.ds(k * L, L)] + g2_v[pl.ds(k * L, L)] + bvec)

    pltpu.sync_copy(out_v, out_hbm.at[pl.ds(base, BPW)])


def kernel(article1_idx, article2_idx, vector_tensor, W, b):
    tab_t = vector_tensor.T
    wcols = W.astype(jnp.float32).reshape(2, D).T
    s1, s2 = _tc_scan(tab_t, tab_t, wcols)
    bias = jnp.broadcast_to(b.astype(jnp.float32), (L,))
    out = _sc_combine(article1_idx.astype(jnp.int32),
                      article2_idx.astype(jnp.int32),
                      s1, s2, bias)
    return out.reshape(B, 1)


# speedup vs baseline: 1.0175x; 1.0017x over previous
import functools

import jax
import jax.numpy as jnp
from jax import lax
from jax.experimental import pallas as pl
from jax.experimental.pallas import tpu as pltpu
from jax.experimental.pallas import tpu_sc as plsc

N = 1000000
B = 16384
D = 64
L = 16
NC, NS = 2, 16
NW = NC * NS
BPW = B // NW
CH = 128
NCH = BPW // CH

CBLK = 102400
GRID = (N + CBLK - 1) // CBLK


NSPLIT = 4
DSUB = D // NSPLIT


def _scan_body(*refs):
    tab_refs, w_ref = refs[:NSPLIT], refs[NSPLIT]
    s1_ref, s2_ref = refs[NSPLIT + 1], refs[NSPLIT + 2]
    wt = w_ref[...]
    dn = (((1,), (0,)), ((), ()))
    res = sum(
        jax.lax.dot_general(wt[k * DSUB:(k + 1) * DSUB].T, tab_refs[k][...],
                            dn, preferred_element_type=jnp.float32)
        for k in range(NSPLIT))
    s1_ref[...] = res[0]
    s2_ref[...] = res[1]


_tc_scan = pl.pallas_call(
    _scan_body,
    grid=(GRID,),
    in_specs=[
        pl.BlockSpec((DSUB, CBLK), functools.partial(
            lambda k, i: (k, i), k))
        for k in range(NSPLIT)
    ] + [
        pl.BlockSpec((D, 2), lambda i: (0, 0)),
    ],
    out_specs=[
        pl.BlockSpec((CBLK,), lambda i: (i,)),
        pl.BlockSpec((CBLK,), lambda i: (i,)),
    ],
    out_shape=[
        jax.ShapeDtypeStruct((N,), jnp.float32),
        jax.ShapeDtypeStruct((N,), jnp.float32),
    ],
)

_mesh = plsc.VectorSubcoreMesh(core_axis_name="c", subcore_axis_name="s")


@functools.partial(
    pl.kernel,
    out_type=jax.ShapeDtypeStruct((B,), jnp.float32),
    mesh=_mesh,
    compiler_params=pltpu.CompilerParams(needs_layout_passes=False),
    scratch_types=[
        pltpu.VMEM((NCH, CH), jnp.int32),
        pltpu.VMEM((NCH, CH), jnp.int32),
        pltpu.VMEM((BPW,), jnp.float32),
        pltpu.VMEM((BPW,), jnp.float32),
        pltpu.VMEM((L,), jnp.float32),
        pltpu.VMEM((BPW,), jnp.float32),
        pltpu.SemaphoreType.DMA,
    ],
)
def _sc_combine(idx1_hbm, idx2_hbm, s1_hbm, s2_hbm, bias_hbm, out_hbm,
                idx1_v, idx2_v, g1_v, g2_v, b_v, out_v, sem):
    wid = lax.axis_index("s") * NC + lax.axis_index("c")
    base = wid * BPW

    pltpu.sync_copy(bias_hbm, b_v)
    for j in range(NCH):
        pltpu.sync_copy(idx1_hbm.at[pl.ds(base + j * CH, CH)], idx1_v.at[j])
        pltpu.sync_copy(idx2_hbm.at[pl.ds(base + j * CH, CH)], idx2_v.at[j])

    copies = []
    for j in range(NCH):
        copies.append(pltpu.async_copy(
            s1_hbm.at[idx1_v.at[j]], g1_v.at[pl.ds(j * CH, CH)], sem))
        copies.append(pltpu.async_copy(
            s2_hbm.at[idx2_v.at[j]], g2_v.at[pl.ds(j * CH, CH)], sem))
    for c in copies:
        c.wait()

    bvec = b_v[pl.ds(0, L)]
    for k in range(BPW // L):
        out_v[pl.ds(k * L, L)] = (
            g1_v[pl.ds(k * L, L)] + g2_v[pl.ds(k * L, L)] + bvec)

    pltpu.sync_copy(out_v, out_hbm.at[pl.ds(base, BPW)])


def kernel(article1_idx, article2_idx, vector_tensor, W, b):
    tab_t = vector_tensor.T
    wcols = W.astype(jnp.float32).reshape(2, D).T
    s1, s2 = _tc_scan(*([tab_t] * NSPLIT), wcols)
    bias = jnp.broadcast_to(b.astype(jnp.float32), (L,))
    out = _sc_combine(article1_idx.astype(jnp.int32),
                      article2_idx.astype(jnp.int32),
                      s1, s2, bias)
    return out.reshape(B, 1)


# speedup vs baseline: 1.0639x; 1.0456x over previous
import functools

import jax
import jax.numpy as jnp
from jax import lax
from jax.experimental import pallas as pl
from jax.experimental.pallas import tpu as pltpu
from jax.experimental.pallas import tpu_sc as plsc

N = 1000000
B = 16384
D = 64
L = 16
NC, NS = 2, 16
NW = NC * NS
BPW = B // NW
CH = 128
NCH = BPW // CH

CBLK = 102400
GRID = (N + CBLK - 1) // CBLK


NSPLIT = 4
DSUB = D // NSPLIT


def _scan_body(*refs):
    tab_refs, w_ref = refs[:NSPLIT], refs[NSPLIT]
    s1_ref, s2_ref = refs[NSPLIT + 1], refs[NSPLIT + 2]
    wt = w_ref[...]
    dn = (((1,), (0,)), ((), ()))
    res = sum(
        jax.lax.dot_general(wt[k * DSUB:(k + 1) * DSUB].T, tab_refs[k][...],
                            dn, preferred_element_type=jnp.float32)
        for k in range(NSPLIT))
    s1_ref[...] = res[0]
    s2_ref[...] = res[1]


_tc_scan = pl.pallas_call(
    _scan_body,
    grid=(GRID,),
    in_specs=[
        pl.BlockSpec((DSUB, CBLK), functools.partial(
            lambda k, i: (k, i), k))
        for k in range(NSPLIT)
    ] + [
        pl.BlockSpec((D, 2), lambda i: (0, 0)),
    ],
    out_specs=[
        pl.BlockSpec((CBLK,), lambda i: (i,)),
        pl.BlockSpec((CBLK,), lambda i: (i,)),
    ],
    out_shape=[
        jax.ShapeDtypeStruct((N,), jnp.float32),
        jax.ShapeDtypeStruct((N,), jnp.float32),
    ],
)

_mesh = plsc.VectorSubcoreMesh(core_axis_name="c", subcore_axis_name="s")


@functools.partial(
    pl.kernel,
    out_type=jax.ShapeDtypeStruct((B,), jnp.float32),
    mesh=_mesh,
    compiler_params=pltpu.CompilerParams(needs_layout_passes=False),
    scratch_types=[
        pltpu.VMEM((NCH, CH), jnp.int32),
        pltpu.VMEM((NCH, CH), jnp.int32),
        pltpu.VMEM((BPW,), jnp.float32),
        pltpu.VMEM((BPW,), jnp.float32),
        pltpu.VMEM((L,), jnp.float32),
        pltpu.VMEM((BPW,), jnp.float32),
        pltpu.SemaphoreType.DMA,
    ],
)
def _sc_combine(idx1_hbm, idx2_hbm, s1_hbm, s2_hbm, bias_hbm, out_hbm,
                idx1_v, idx2_v, g1_v, g2_v, b_v, out_v, sem):
    wid = lax.axis_index("s") * NC + lax.axis_index("c")
    base = wid * BPW

    stage = [pltpu.async_copy(bias_hbm, b_v, sem)]
    for j in range(NCH):
        stage.append(pltpu.async_copy(
            idx1_hbm.at[pl.ds(base + j * CH, CH)], idx1_v.at[j], sem))
        stage.append(pltpu.async_copy(
            idx2_hbm.at[pl.ds(base + j * CH, CH)], idx2_v.at[j], sem))
    for c in stage:
        c.wait()

    copies = []
    for j in range(NCH):
        copies.append(pltpu.async_copy(
            s1_hbm.at[idx1_v.at[j]], g1_v.at[pl.ds(j * CH, CH)], sem))
        copies.append(pltpu.async_copy(
            s2_hbm.at[idx2_v.at[j]], g2_v.at[pl.ds(j * CH, CH)], sem))
    for c in copies:
        c.wait()

    bvec = b_v[pl.ds(0, L)]
    for k in range(BPW // L):
        out_v[pl.ds(k * L, L)] = (
            g1_v[pl.ds(k * L, L)] + g2_v[pl.ds(k * L, L)] + bvec)

    pltpu.sync_copy(out_v, out_hbm.at[pl.ds(base, BPW)])


def kernel(article1_idx, article2_idx, vector_tensor, W, b):
    tab_t = vector_tensor.T
    wcols = W.astype(jnp.float32).reshape(2, D).T
    s1, s2 = _tc_scan(*([tab_t] * NSPLIT), wcols)
    bias = jnp.broadcast_to(b.astype(jnp.float32), (L,))
    out = _sc_combine(article1_idx.astype(jnp.int32),
                      article2_idx.astype(jnp.int32),
                      s1, s2, bias)
    return out.reshape(B, 1)
